# baseline (speedup 1.0000x reference)
import functools

import jax
import jax.numpy as jnp
from jax import lax
from jax.experimental import pallas as pl
from jax.experimental.pallas import tpu as pltpu
from jax.experimental.pallas import tpu_sc as plsc

_B = 4096
_SD = 200 * 64
_LANES = 16
_NW = 32
_CH = 4
_NB = 4
_UNROLL = 8

_SC_ROWS = 5632
_TC_ROWS = _SD - _SC_ROWS
_TC_BLK = 512


def _make_sc_kernel(sc_rows):
    rpw = sc_rows // _NW
    n_outer = rpw // _CH // _NB
    mesh = plsc.VectorSubcoreMesh(core_axis_name="c", subcore_axis_name="s")

    @functools.partial(
        pl.kernel,
        mesh=mesh,
        out_type=jax.ShapeDtypeStruct((sc_rows, _B), jnp.float32),
        scratch_types=[
            pltpu.VMEM((rpw * _LANES,), jnp.float32),
            pltpu.VMEM((_NB, _CH, _B), jnp.float32),
            pltpu.SemaphoreType.DMA((_NB,)),
            pltpu.SemaphoreType.DMA((_NB,)),
        ],
    )
    def sc_add(x_hbm, posb_hbm, out_hbm, posv, buf, in_sems, out_sems):
        wid = lax.axis_index("s") * 2 + lax.axis_index("c")
        base = wid * rpw

        pltpu.sync_copy(
            posb_hbm.at[pl.ds(base * _LANES, rpw * _LANES)], posv
        )

        for b in range(_NB):
            pltpu.async_copy(
                x_hbm.at[pl.ds(base + b * _CH, _CH)], buf.at[b], in_sems.at[b]
            )

        def outer(i, carry):
            for b in range(_NB):
                k = i * _NB + b
                row0 = base + k * _CH
                pltpu.make_async_copy(
                    x_hbm.at[pl.ds(row0, _CH)], buf.at[b], in_sems.at[b]
                ).wait()

                for r in range(_CH):
                    rowref = buf.at[b, r]
                    psplat = posv[pl.ds((k * _CH + r) * _LANES, _LANES)]

                    @plsc.parallel_loop(0, _B, step=_LANES, unroll=_UNROLL)
                    def _add(c):
                        sl = pl.ds(c, _LANES)
                        rowref[sl] = rowref[sl] + psplat

                pltpu.async_copy(
                    buf.at[b], out_hbm.at[pl.ds(row0, _CH)], out_sems.at[b]
                )

                @pl.when(i < n_outer - 1)
                def _prefetch():
                    pltpu.make_async_copy(
                        buf.at[b], out_hbm.at[pl.ds(row0, _CH)], out_sems.at[b]
                    ).wait()
                    pltpu.async_copy(
                        x_hbm.at[pl.ds(row0 + _NB * _CH, _CH)],
                        buf.at[b],
                        in_sems.at[b],
                    )

            return carry

        lax.fori_loop(0, n_outer, outer, 0)

        for b in range(_NB):
            row0 = base + ((n_outer - 1) * _NB + b) * _CH
            pltpu.make_async_copy(
                buf.at[b], out_hbm.at[pl.ds(row0, _CH)], out_sems.at[b]
            ).wait()

    return sc_add


def _tc_add_body(x_ref, pos_ref, out_ref):
    out_ref[...] = x_ref[...] + pos_ref[...]


def _tc_tail(xt, post):
    off = _SC_ROWS // _TC_BLK
    return pl.pallas_call(
        _tc_add_body,
        grid=(_TC_ROWS // _TC_BLK,),
        in_specs=[
            pl.BlockSpec((_TC_BLK, _B), lambda i: (i + off, 0)),
            pl.BlockSpec((_TC_BLK, 1), lambda i: (i + off, 0)),
        ],
        out_specs=pl.BlockSpec((_TC_BLK, _B), lambda i: (i + off, 0)),
        out_shape=jax.ShapeDtypeStruct((_SD, _B), jnp.float32),
        compiler_params=pltpu.CompilerParams(
            dimension_semantics=("arbitrary",),
        ),
    )(xt, post)


def kernel(x, pos_table):
    B, S, D = x.shape
    xt = x.transpose(1, 2, 0).reshape(S * D, B)
    pos_flat = pos_table[:S].reshape(S * D)
    posb = jnp.repeat(pos_flat, _LANES)
    sc_out = _make_sc_kernel(_SC_ROWS)(xt, posb)
    tc_full = _tc_tail(xt, pos_flat.reshape(S * D, 1))
    out_t = lax.dynamic_update_slice(tc_full, sc_out, (0, 0))
    return out_t.reshape(S, D, B).transpose(2, 0, 1)


# speedup vs baseline: 1.3294x; 1.3294x over previous
import functools

import jax
import jax.numpy as jnp
from jax import lax
from jax.experimental import pallas as pl
from jax.experimental.pallas import tpu as pltpu
from jax.experimental.pallas import tpu_sc as plsc

_B = 4096
_SD = 200 * 64
_LANES = 16
_NW = 32
_CH = 4
_NB = 4
_UNROLL = 8

_SC_ROWS = 5632
_TC_ROWS = _SD - _SC_ROWS
_TC_BLK = 512


def _make_sc_kernel(sc_rows):
    rpw = sc_rows // _NW
    n_outer = rpw // _CH // _NB
    mesh = plsc.VectorSubcoreMesh(core_axis_name="c", subcore_axis_name="s")

    @functools.partial(
        pl.kernel,
        mesh=mesh,
        out_type=jax.ShapeDtypeStruct((_SD, _B), jnp.float32),
        scratch_types=[
            pltpu.VMEM((rpw * _LANES,), jnp.float32),
            pltpu.VMEM((_NB, _CH, _B), jnp.float32),
            pltpu.SemaphoreType.DMA((_NB,)),
            pltpu.SemaphoreType.DMA((_NB,)),
        ],
    )
    def sc_add(x_hbm, posb_hbm, out_hbm, posv, buf, in_sems, out_sems):
        wid = lax.axis_index("s") * 2 + lax.axis_index("c")
        base = wid * rpw

        pltpu.sync_copy(
            posb_hbm.at[pl.ds(base * _LANES, rpw * _LANES)], posv
        )

        for b in range(_NB):
            pltpu.async_copy(
                x_hbm.at[pl.ds(base + b * _CH, _CH)], buf.at[b], in_sems.at[b]
            )

        def outer(i, carry):
            for b in range(_NB):
                k = i * _NB + b
                row0 = base + k * _CH
                pltpu.make_async_copy(
                    x_hbm.at[pl.ds(row0, _CH)], buf.at[b], in_sems.at[b]
                ).wait()

                for r in range(_CH):
                    rowref = buf.at[b, r]
                    psplat = posv[pl.ds((k * _CH + r) * _LANES, _LANES)]

                    @plsc.parallel_loop(0, _B, step=_LANES, unroll=_UNROLL)
                    def _add(c):
                        sl = pl.ds(c, _LANES)
                        rowref[sl] = rowref[sl] + psplat

                pltpu.async_copy(
                    buf.at[b], out_hbm.at[pl.ds(row0, _CH)], out_sems.at[b]
                )

                @pl.when(i < n_outer - 1)
                def _prefetch():
                    pltpu.make_async_copy(
                        buf.at[b], out_hbm.at[pl.ds(row0, _CH)], out_sems.at[b]
                    ).wait()
                    pltpu.async_copy(
                        x_hbm.at[pl.ds(row0 + _NB * _CH, _CH)],
                        buf.at[b],
                        in_sems.at[b],
                    )

            return carry

        lax.fori_loop(0, n_outer, outer, 0)

        for b in range(_NB):
            row0 = base + ((n_outer - 1) * _NB + b) * _CH
            pltpu.make_async_copy(
                buf.at[b], out_hbm.at[pl.ds(row0, _CH)], out_sems.at[b]
            ).wait()

    return sc_add


def _tc_add_body(x_ref, pos_ref, out_ref):
    out_ref[...] = x_ref[...] + pos_ref[...]


def _tc_tail_body(x_ref, pos_ref, sc_ref, out_ref):
    del sc_ref
    out_ref[...] = x_ref[...] + pos_ref[...]


def _tc_tail(xt, post, sc_full):
    off = _SC_ROWS // _TC_BLK
    return pl.pallas_call(
        _tc_tail_body,
        grid=(_TC_ROWS // _TC_BLK,),
        in_specs=[
            pl.BlockSpec((_TC_BLK, _B), lambda i: (i + off, 0)),
            pl.BlockSpec((_TC_BLK, 1), lambda i: (i + off, 0)),
            pl.BlockSpec(memory_space=pl.ANY),
        ],
        out_specs=pl.BlockSpec((_TC_BLK, _B), lambda i: (i + off, 0)),
        out_shape=jax.ShapeDtypeStruct((_SD, _B), jnp.float32),
        input_output_aliases={2: 0},
        compiler_params=pltpu.CompilerParams(
            dimension_semantics=("arbitrary",),
        ),
    )(xt, post, sc_full)


def kernel(x, pos_table):
    B, S, D = x.shape
    xt = x.transpose(1, 2, 0).reshape(S * D, B)
    pos_flat = pos_table[:S].reshape(S * D)
    posb = jnp.repeat(pos_flat, _LANES)
    sc_full = _make_sc_kernel(_SC_ROWS)(xt, posb)
    out_t = _tc_tail(xt, pos_flat.reshape(S * D, 1), sc_full)
    return out_t.reshape(S, D, B).transpose(2, 0, 1)


# speedup vs baseline: 1.3503x; 1.0158x over previous
import functools

import jax
import jax.numpy as jnp
from jax import lax
from jax.experimental import pallas as pl
from jax.experimental.pallas import tpu as pltpu
from jax.experimental.pallas import tpu_sc as plsc

_B = 4096
_SD = 200 * 64
_LANES = 16
_NW = 32
_CH = 4
_NB = 4
_UNROLL = 8

_SC_ROWS = 3072
_TC_ROWS = _SD - _SC_ROWS
_TC_BLK = 512


def _make_sc_kernel(sc_rows):
    rpw = sc_rows // _NW
    n_outer = rpw // _CH // _NB
    mesh = plsc.VectorSubcoreMesh(core_axis_name="c", subcore_axis_name="s")

    @functools.partial(
        pl.kernel,
        mesh=mesh,
        out_type=jax.ShapeDtypeStruct((_SD, _B), jnp.float32),
        scratch_types=[
            pltpu.VMEM((rpw * _LANES,), jnp.float32),
            pltpu.VMEM((_NB, _CH, _B), jnp.float32),
            pltpu.SemaphoreType.DMA((_NB,)),
            pltpu.SemaphoreType.DMA((_NB,)),
        ],
    )
    def sc_add(x_hbm, posb_hbm, out_hbm, posv, buf, in_sems, out_sems):
        wid = lax.axis_index("s") * 2 + lax.axis_index("c")
        base = wid * rpw

        pltpu.sync_copy(
            posb_hbm.at[pl.ds(base * _LANES, rpw * _LANES)], posv
        )

        for b in range(_NB):
            pltpu.async_copy(
                x_hbm.at[pl.ds(base + b * _CH, _CH)], buf.at[b], in_sems.at[b]
            )

        def outer(i, carry):
            for b in range(_NB):
                k = i * _NB + b
                row0 = base + k * _CH
                pltpu.make_async_copy(
                    x_hbm.at[pl.ds(row0, _CH)], buf.at[b], in_sems.at[b]
                ).wait()

                for r in range(_CH):
                    rowref = buf.at[b, r]
                    psplat = posv[pl.ds((k * _CH + r) * _LANES, _LANES)]

                    @plsc.parallel_loop(0, _B, step=_LANES, unroll=_UNROLL)
                    def _add(c):
                        sl = pl.ds(c, _LANES)
                        rowref[sl] = rowref[sl] + psplat

                pltpu.async_copy(
                    buf.at[b], out_hbm.at[pl.ds(row0, _CH)], out_sems.at[b]
                )

                @pl.when(i < n_outer - 1)
                def _prefetch():
                    pltpu.make_async_copy(
                        buf.at[b], out_hbm.at[pl.ds(row0, _CH)], out_sems.at[b]
                    ).wait()
                    pltpu.async_copy(
                        x_hbm.at[pl.ds(row0 + _NB * _CH, _CH)],
                        buf.at[b],
                        in_sems.at[b],
                    )

            return carry

        lax.fori_loop(0, n_outer, outer, 0)

        for b in range(_NB):
            row0 = base + ((n_outer - 1) * _NB + b) * _CH
            pltpu.make_async_copy(
                buf.at[b], out_hbm.at[pl.ds(row0, _CH)], out_sems.at[b]
            ).wait()

    return sc_add


def _tc_add_body(x_ref, pos_ref, out_ref):
    out_ref[...] = x_ref[...] + pos_ref[...]


def _tc_tail_body(x_ref, pos_ref, sc_ref, out_ref):
    del sc_ref
    out_ref[...] = x_ref[...] + pos_ref[...]


def _tc_tail(xt, post, sc_full):
    off = _SC_ROWS // _TC_BLK
    return pl.pallas_call(
        _tc_tail_body,
        grid=(_TC_ROWS // _TC_BLK,),
        in_specs=[
            pl.BlockSpec((_TC_BLK, _B), lambda i: (i + off, 0)),
            pl.BlockSpec((_TC_BLK, 1), lambda i: (i + off, 0)),
            pl.BlockSpec(memory_space=pl.ANY),
        ],
        out_specs=pl.BlockSpec((_TC_BLK, _B), lambda i: (i + off, 0)),
        out_shape=jax.ShapeDtypeStruct((_SD, _B), jnp.float32),
        input_output_aliases={2: 0},
        compiler_params=pltpu.CompilerParams(
            dimension_semantics=("arbitrary",),
        ),
    )(xt, post, sc_full)


def kernel(x, pos_table):
    B, S, D = x.shape
    xt = x.transpose(1, 2, 0).reshape(S * D, B)
    pos_flat = pos_table[:S].reshape(S * D)
    posb = jnp.repeat(pos_flat, _LANES)
    sc_full = _make_sc_kernel(_SC_ROWS)(xt, posb)
    out_t = _tc_tail(xt, pos_flat.reshape(S * D, 1), sc_full)
    return out_t.reshape(S, D, B).transpose(2, 0, 1)


# speedup vs baseline: 1.3511x; 1.0006x over previous
import functools

import jax
import jax.numpy as jnp
from jax import lax
from jax.experimental import pallas as pl
from jax.experimental.pallas import tpu as pltpu
from jax.experimental.pallas import tpu_sc as plsc

_B = 4096
_SD = 200 * 64
_LANES = 16
_NW = 32
_CH = 4
_NB = 4
_UNROLL = 8

_SC_ROWS = 3072
_TC_ROWS = _SD - _SC_ROWS
_TC_BLK = 512


def _make_sc_kernel(sc_rows):
    rpw = sc_rows // _NW
    n_outer = rpw // _CH // _NB
    mesh = plsc.VectorSubcoreMesh(core_axis_name="c", subcore_axis_name="s")

    @functools.partial(
        pl.kernel,
        mesh=mesh,
        out_type=jax.ShapeDtypeStruct((_SD, _B), jnp.float32),
        scratch_types=[
            pltpu.VMEM((rpw * _LANES,), jnp.float32),
            pltpu.VMEM((_NB, _CH, _B), jnp.float32),
            pltpu.SemaphoreType.DMA((_NB,)),
            pltpu.SemaphoreType.DMA((_NB,)),
        ],
    )
    def sc_add(x_hbm, posb_hbm, out_hbm, posv, buf, in_sems, out_sems):
        wid = lax.axis_index("s") * 2 + lax.axis_index("c")
        base = wid * rpw

        pltpu.sync_copy(
            posb_hbm.at[pl.ds(base * _LANES, rpw * _LANES)], posv
        )

        for b in range(_NB):
            pltpu.async_copy(
                x_hbm.at[pl.ds(base + b * _CH, _CH)], buf.at[b], in_sems.at[b]
            )

        def outer(i, carry):
            for b in range(_NB):
                k = i * _NB + b
                row0 = base + k * _CH
                pltpu.make_async_copy(
                    x_hbm.at[pl.ds(row0, _CH)], buf.at[b], in_sems.at[b]
                ).wait()

                for r in range(_CH):
                    rowref = buf.at[b, r]
                    psplat = posv[pl.ds((k * _CH + r) * _LANES, _LANES)]

                    @plsc.parallel_loop(0, _B, step=_LANES, unroll=_UNROLL)
                    def _add(c):
                        sl = pl.ds(c, _LANES)
                        rowref[sl] = rowref[sl] + psplat

                pltpu.async_copy(
                    buf.at[b], out_hbm.at[pl.ds(row0, _CH)], out_sems.at[b]
                )

                @pl.when(i < n_outer - 1)
                def _prefetch():
                    pltpu.make_async_copy(
                        buf.at[b], out_hbm.at[pl.ds(row0, _CH)], out_sems.at[b]
                    ).wait()
                    pltpu.async_copy(
                        x_hbm.at[pl.ds(row0 + _NB * _CH, _CH)],
                        buf.at[b],
                        in_sems.at[b],
                    )

            return carry

        lax.fori_loop(0, n_outer, outer, 0)

        for b in range(_NB):
            row0 = base + ((n_outer - 1) * _NB + b) * _CH
            pltpu.make_async_copy(
                buf.at[b], out_hbm.at[pl.ds(row0, _CH)], out_sems.at[b]
            ).wait()

    return sc_add


def _tc_tail_body(x_ref, pos_ref, sc_ref, out_ref):
    del sc_ref
    out_ref[...] = x_ref[...] + pos_ref[...]


def _tc_tail(xt, post, sc_full):
    off = _SC_ROWS // _TC_BLK
    return pl.pallas_call(
        _tc_tail_body,
        grid=(_TC_ROWS // _TC_BLK,),
        in_specs=[
            pl.BlockSpec((_TC_BLK, _B), lambda i: (i + off, 0)),
            pl.BlockSpec((_TC_BLK, 1), lambda i: (i + off, 0)),
            pl.BlockSpec(memory_space=pl.ANY),
        ],
        out_specs=pl.BlockSpec((_TC_BLK, _B), lambda i: (i + off, 0)),
        out_shape=jax.ShapeDtypeStruct((_SD, _B), jnp.float32),
        input_output_aliases={2: 0},
        compiler_params=pltpu.CompilerParams(
            dimension_semantics=("arbitrary",),
        ),
    )(xt, post, sc_full)


def kernel(x, pos_table):
    B, S, D = x.shape
    xt = x.transpose(1, 2, 0).reshape(S * D, B)
    pos_flat = pos_table[:S].reshape(S * D)
    posb = jnp.repeat(pos_flat, _LANES)
    sc_full = _make_sc_kernel(_SC_ROWS)(xt, posb)
    out_t = _tc_tail(xt, pos_flat.reshape(S * D, 1), sc_full)
    return out_t.reshape(S, D, B).transpose(2, 0, 1)
